# Initial kernel scaffold; baseline (speedup 1.0000x reference)
import jax
import jax.numpy as jnp
from jax.experimental import pallas as pl


def kernel(x, edge_index, batch, eps1, Wa1, ba1, g1, be1, Wb1, bb1, eps2, Wa2, ba2, g2, be2, Wb2, bb2, eps3, Wa3, ba3, g3, be3, Wb3, bb3, eps4, Wa4, ba4, g4, be4, Wb4, bb4, Wl, bl):
    raise NotImplementedError("write your pallas kernel here")


# speedup vs baseline: 7.2351x; 7.2351x over previous
import functools

import jax
import jax.numpy as jnp
from jax import lax
from jax.experimental import pallas as pl
from jax.experimental.pallas import tpu as pltpu
from jax.experimental.pallas import tpu_sc as plsc

N = 10000
E = 320000
G = 64
D = 32

NC = 2
NS = 16
NW = NC * NS
CH = 128
PERW = 80
NCHT = NW * PERW
EPAD = NCHT * CH - E
RPT = 632
NP = RPT * NS


def _segsum_body(h_hbm, src_hbm, dst_hbm, zero_hbm, out_hbm,
                 acc_sh, src_v, dst_v, rows_a, sem):
    c = lax.axis_index("c")
    s = lax.axis_index("s")
    wid = c * NS + s
    cbase = wid * PERW

    pltpu.sync_copy(zero_hbm, acc_sh.at[pl.ds(s * RPT, RPT)])

    pltpu.sync_copy(src_hbm.at[pl.ds(cbase, PERW)], src_v)
    pltpu.sync_copy(dst_hbm.at[pl.ds(cbase, PERW)], dst_v)

    plsc.subcore_barrier()

    def chunk(i, carry):
        pltpu.async_copy(h_hbm.at[src_v.at[i]], rows_a, sem).wait()
        pltpu.sync_copy(rows_a, acc_sh.at[dst_v.at[i]], add=True)
        return carry

    lax.fori_loop(0, PERW, chunk, 0, unroll=False)

    plsc.subcore_barrier()

    pltpu.sync_copy(acc_sh.at[pl.ds(s * RPT, RPT)],
                    out_hbm.at[c, pl.ds(s * RPT, RPT)])


@functools.partial(jax.jit, static_argnums=())
def _segsum(h, src, dst, zero_blk):
    mesh = plsc.VectorSubcoreMesh(
        core_axis_name="c", subcore_axis_name="s",
        num_cores=NC, num_subcores=NS)
    fn = pl.kernel(
        _segsum_body,
        out_type=jax.ShapeDtypeStruct((NC, NP, D), jnp.float32),
        mesh=mesh,
        scratch_types=[
            pltpu.VMEM_SHARED((NP, D), jnp.float32),
            pltpu.VMEM((PERW, CH), jnp.int32),
            pltpu.VMEM((PERW, CH), jnp.int32),
            pltpu.VMEM((CH, D), jnp.float32),
            pltpu.SemaphoreType.DMA,
        ],
        compiler_params=pltpu.CompilerParams(use_tc_tiling_on_sc=False),
    )
    return fn(h, src, dst, zero_blk)


def _proj_body(x_ref, w_ref, o_ref):
    o_ref[...] = jnp.dot(x_ref[...], w_ref[...],
                         preferred_element_type=jnp.float32)


def _proj(x, w, dout):
    return pl.pallas_call(
        _proj_body,
        out_shape=jax.ShapeDtypeStruct((x.shape[0], dout), jnp.float32),
    )(x, w)


def _norm_relu(u, g_ref, be_ref):
    mu = jnp.mean(u, axis=0, keepdims=True)
    var = jnp.mean((u - mu) ** 2, axis=0, keepdims=True)
    un = (u - mu) / jnp.sqrt(var + 1e-5) * g_ref[...] + be_ref[...]
    return jnp.maximum(un, 0.0)


def _mlp1_body(y_ref, agg_ref, eps_ref, ba_ref, g_ref, be_ref, wb_ref,
               bb_ref, o_ref):
    y = y_ref[...]
    u = (1.0 + eps_ref[0, 0]) * y + agg_ref[0, :N] + agg_ref[1, :N] + ba_ref[...]
    h = _norm_relu(u, g_ref, be_ref)
    o_ref[...] = jnp.dot(h, wb_ref[...],
                         preferred_element_type=jnp.float32) + bb_ref[...]


def _mlp1(y, agg, eps, ba, g, be, Wb, bb):
    return pl.pallas_call(
        _mlp1_body,
        out_shape=jax.ShapeDtypeStruct((N, Wb.shape[1]), jnp.float32),
    )(y, agg, eps.reshape(1, 1), ba.reshape(1, -1), g.reshape(1, -1),
      be.reshape(1, -1), Wb, bb.reshape(1, -1))


def _mlp_body(h_ref, agg_ref, eps_ref, wa_ref, ba_ref, g_ref, be_ref,
              wb_ref, bb_ref, o_ref):
    t = (1.0 + eps_ref[0, 0]) * h_ref[...] + agg_ref[0, :N] + agg_ref[1, :N]
    y = jnp.dot(t, wa_ref[...],
                preferred_element_type=jnp.float32) + ba_ref[...]
    h = _norm_relu(y, g_ref, be_ref)
    o_ref[...] = jnp.dot(h, wb_ref[...],
                         preferred_element_type=jnp.float32) + bb_ref[...]


def _mlp(h, agg, eps, Wa, ba, g, be, Wb, bb):
    return pl.pallas_call(
        _mlp_body,
        out_shape=jax.ShapeDtypeStruct((N, Wb.shape[1]), jnp.float32),
    )(h, agg, eps.reshape(1, 1), Wa, ba.reshape(1, -1), g.reshape(1, -1),
      be.reshape(1, -1), Wb, bb.reshape(1, -1))


def _final_body(h_ref, agg_ref, eps_ref, wa_ref, ba_ref, g_ref, be_ref,
                wb_ref, bb_ref, batch_ref, wl_ref, bl_ref, o_ref):
    t = (1.0 + eps_ref[0, 0]) * h_ref[...] + agg_ref[0, :N] + agg_ref[1, :N]
    y = jnp.dot(t, wa_ref[...],
                preferred_element_type=jnp.float32) + ba_ref[...]
    h = _norm_relu(y, g_ref, be_ref)
    h4 = jnp.dot(h, wb_ref[...],
                 preferred_element_type=jnp.float32) + bb_ref[...]
    gids = lax.broadcasted_iota(jnp.int32, (N, G), 1)
    onehot = (batch_ref[...] == gids).astype(jnp.float32)
    sums = lax.dot_general(onehot, h4, (((0,), (0,)), ((), ())),
                           preferred_element_type=jnp.float32)
    counts = jnp.sum(onehot, axis=0)[:, None]
    pooled = sums / jnp.maximum(counts, 1.0)
    logit = jnp.dot(pooled, wl_ref[...],
                    preferred_element_type=jnp.float32) + bl_ref[...]
    o_ref[...] = jax.nn.sigmoid(logit)


def _final(h, agg, eps, Wa, ba, g, be, Wb, bb, batch, Wl, bl):
    return pl.pallas_call(
        _final_body,
        out_shape=jax.ShapeDtypeStruct((G, 1), jnp.float32),
    )(h, agg, eps.reshape(1, 1), Wa, ba.reshape(1, -1), g.reshape(1, -1),
      be.reshape(1, -1), Wb, bb.reshape(1, -1), batch.reshape(N, 1), Wl,
      bl.reshape(1, -1))


def kernel(x, edge_index, batch, eps1, Wa1, ba1, g1, be1, Wb1, bb1,
           eps2, Wa2, ba2, g2, be2, Wb2, bb2,
           eps3, Wa3, ba3, g3, be3, Wb3, bb3,
           eps4, Wa4, ba4, g4, be4, Wb4, bb4, Wl, bl):
    pad_src = jnp.zeros((EPAD,), jnp.int32)
    pad_dst = jnp.full((EPAD,), N, jnp.int32)
    src = jnp.concatenate([edge_index[0], pad_src]).reshape(NCHT, CH)
    dst = jnp.concatenate([edge_index[1], pad_dst]).reshape(NCHT, CH)
    zero_blk = jnp.zeros((RPT, D), jnp.float32)

    y1 = _proj(x, Wa1, D)
    a1 = _segsum(y1, src, dst, zero_blk)
    h1 = _mlp1(y1, a1, eps1, ba1, g1, be1, Wb1, bb1)

    a2 = _segsum(h1, src, dst, zero_blk)
    h2 = _mlp(h1, a2, eps2, Wa2, ba2, g2, be2, Wb2, bb2)

    a3 = _segsum(h2, src, dst, zero_blk)
    h3 = _mlp(h2, a3, eps3, Wa3, ba3, g3, be3, Wb3, bb3)

    a4 = _segsum(h3, src, dst, zero_blk)
    return _final(h3, a4, eps4, Wa4, ba4, g4, be4, Wb4, bb4, batch, Wl, bl)


# speedup vs baseline: 8.7124x; 1.2042x over previous
import functools

import jax
import jax.numpy as jnp
from jax import lax
from jax.experimental import pallas as pl
from jax.experimental.pallas import tpu as pltpu
from jax.experimental.pallas import tpu_sc as plsc

N = 10000
E = 320000
G = 64
D = 32

NC = 2
NS = 16
NW = NC * NS
CH = 128
PERW = 80
NCHT = NW * PERW
EPAD = NCHT * CH - E
RPT = 632
KB = 8
NP = RPT * NS


def _segsum_body(h_hbm, src_hbm, dst_hbm, zero_hbm, out_hbm,
                 acc_sh, src_v, dst_v, rows_a, gsem, sem):
    c = lax.axis_index("c")
    s = lax.axis_index("s")
    wid = c * NS + s
    cbase = wid * PERW

    pltpu.sync_copy(zero_hbm, acc_sh.at[pl.ds(s * RPT, RPT)])

    pltpu.sync_copy(src_hbm.at[pl.ds(cbase, PERW)], src_v)
    pltpu.sync_copy(dst_hbm.at[pl.ds(cbase, PERW)], dst_v)

    plsc.subcore_barrier()

    def group(j, carry):
        base = j * KB
        gds = [
            pltpu.async_copy(h_hbm.at[src_v.at[base + b]], rows_a.at[b],
                             gsem.at[b])
            for b in range(KB)
        ]
        sds = []
        for b in range(KB):
            gds[b].wait()
            sds.append(
                pltpu.async_copy(rows_a.at[b], acc_sh.at[dst_v.at[base + b]],
                                 sem, add=True))
        for sd in sds:
            sd.wait()
        return carry

    lax.fori_loop(0, PERW // KB, group, 0, unroll=False)

    plsc.subcore_barrier()

    pltpu.sync_copy(acc_sh.at[pl.ds(s * RPT, RPT)],
                    out_hbm.at[c, pl.ds(s * RPT, RPT)])


@functools.partial(jax.jit, static_argnums=())
def _segsum(h, src, dst, zero_blk):
    mesh = plsc.VectorSubcoreMesh(
        core_axis_name="c", subcore_axis_name="s",
        num_cores=NC, num_subcores=NS)
    fn = pl.kernel(
        _segsum_body,
        out_type=jax.ShapeDtypeStruct((NC, NP, D), jnp.float32),
        mesh=mesh,
        scratch_types=[
            pltpu.VMEM_SHARED((NP, D), jnp.float32),
            pltpu.VMEM((PERW, CH), jnp.int32),
            pltpu.VMEM((PERW, CH), jnp.int32),
            pltpu.VMEM((KB, CH, D), jnp.float32),
            pltpu.SemaphoreType.DMA((KB,)),
            pltpu.SemaphoreType.DMA,
        ],
        compiler_params=pltpu.CompilerParams(use_tc_tiling_on_sc=False),
    )
    return fn(h, src, dst, zero_blk)


def _proj_body(x_ref, w_ref, o_ref):
    o_ref[...] = jnp.dot(x_ref[...], w_ref[...],
                         preferred_element_type=jnp.float32)


def _proj(x, w, dout):
    return pl.pallas_call(
        _proj_body,
        out_shape=jax.ShapeDtypeStruct((x.shape[0], dout), jnp.float32),
    )(x, w)


def _norm_relu(u, g_ref, be_ref):
    mu = jnp.mean(u, axis=0, keepdims=True)
    var = jnp.mean((u - mu) ** 2, axis=0, keepdims=True)
    un = (u - mu) / jnp.sqrt(var + 1e-5) * g_ref[...] + be_ref[...]
    return jnp.maximum(un, 0.0)


def _mlp1_body(y_ref, agg_ref, eps_ref, ba_ref, g_ref, be_ref, wb_ref,
               bb_ref, o_ref):
    y = y_ref[...]
    u = (1.0 + eps_ref[0, 0]) * y + agg_ref[0, :N] + agg_ref[1, :N] + ba_ref[...]
    h = _norm_relu(u, g_ref, be_ref)
    o_ref[...] = jnp.dot(h, wb_ref[...],
                         preferred_element_type=jnp.float32) + bb_ref[...]


def _mlp1(y, agg, eps, ba, g, be, Wb, bb):
    return pl.pallas_call(
        _mlp1_body,
        out_shape=jax.ShapeDtypeStruct((N, Wb.shape[1]), jnp.float32),
    )(y, agg, eps.reshape(1, 1), ba.reshape(1, -1), g.reshape(1, -1),
      be.reshape(1, -1), Wb, bb.reshape(1, -1))


def _mlp_body(h_ref, agg_ref, eps_ref, wa_ref, ba_ref, g_ref, be_ref,
              wb_ref, bb_ref, o_ref):
    t = (1.0 + eps_ref[0, 0]) * h_ref[...] + agg_ref[0, :N] + agg_ref[1, :N]
    y = jnp.dot(t, wa_ref[...],
                preferred_element_type=jnp.float32) + ba_ref[...]
    h = _norm_relu(y, g_ref, be_ref)
    o_ref[...] = jnp.dot(h, wb_ref[...],
                         preferred_element_type=jnp.float32) + bb_ref[...]


def _mlp(h, agg, eps, Wa, ba, g, be, Wb, bb):
    return pl.pallas_call(
        _mlp_body,
        out_shape=jax.ShapeDtypeStruct((N, Wb.shape[1]), jnp.float32),
    )(h, agg, eps.reshape(1, 1), Wa, ba.reshape(1, -1), g.reshape(1, -1),
      be.reshape(1, -1), Wb, bb.reshape(1, -1))


def _final_body(h_ref, agg_ref, eps_ref, wa_ref, ba_ref, g_ref, be_ref,
                wb_ref, bb_ref, batch_ref, wl_ref, bl_ref, o_ref):
    t = (1.0 + eps_ref[0, 0]) * h_ref[...] + agg_ref[0, :N] + agg_ref[1, :N]
    y = jnp.dot(t, wa_ref[...],
                preferred_element_type=jnp.float32) + ba_ref[...]
    h = _norm_relu(y, g_ref, be_ref)
    h4 = jnp.dot(h, wb_ref[...],
                 preferred_element_type=jnp.float32) + bb_ref[...]
    gids = lax.broadcasted_iota(jnp.int32, (N, G), 1)
    onehot = (batch_ref[...] == gids).astype(jnp.float32)
    sums = lax.dot_general(onehot, h4, (((0,), (0,)), ((), ())),
                           preferred_element_type=jnp.float32)
    counts = jnp.sum(onehot, axis=0)[:, None]
    pooled = sums / jnp.maximum(counts, 1.0)
    logit = jnp.dot(pooled, wl_ref[...],
                    preferred_element_type=jnp.float32) + bl_ref[...]
    o_ref[...] = jax.nn.sigmoid(logit)


def _final(h, agg, eps, Wa, ba, g, be, Wb, bb, batch, Wl, bl):
    return pl.pallas_call(
        _final_body,
        out_shape=jax.ShapeDtypeStruct((G, 1), jnp.float32),
    )(h, agg, eps.reshape(1, 1), Wa, ba.reshape(1, -1), g.reshape(1, -1),
      be.reshape(1, -1), Wb, bb.reshape(1, -1), batch.reshape(N, 1), Wl,
      bl.reshape(1, -1))


def kernel(x, edge_index, batch, eps1, Wa1, ba1, g1, be1, Wb1, bb1,
           eps2, Wa2, ba2, g2, be2, Wb2, bb2,
           eps3, Wa3, ba3, g3, be3, Wb3, bb3,
           eps4, Wa4, ba4, g4, be4, Wb4, bb4, Wl, bl):
    pad_src = jnp.zeros((EPAD,), jnp.int32)
    pad_dst = jnp.full((EPAD,), N, jnp.int32)
    src = jnp.concatenate([edge_index[0], pad_src]).reshape(NCHT, CH)
    dst = jnp.concatenate([edge_index[1], pad_dst]).reshape(NCHT, CH)
    zero_blk = jnp.zeros((RPT, D), jnp.float32)

    y1 = _proj(x, Wa1, D)
    a1 = _segsum(y1, src, dst, zero_blk)
    h1 = _mlp1(y1, a1, eps1, ba1, g1, be1, Wb1, bb1)

    a2 = _segsum(h1, src, dst, zero_blk)
    h2 = _mlp(h1, a2, eps2, Wa2, ba2, g2, be2, Wb2, bb2)

    a3 = _segsum(h2, src, dst, zero_blk)
    h3 = _mlp(h2, a3, eps3, Wa3, ba3, g3, be3, Wb3, bb3)

    a4 = _segsum(h3, src, dst, zero_blk)
    return _final(h3, a4, eps4, Wa4, ba4, g4, be4, Wb4, bb4, batch, Wl, bl)


# speedup vs baseline: 8.8693x; 1.0180x over previous
import functools

import jax
import jax.numpy as jnp
from jax import lax
from jax.experimental import pallas as pl
from jax.experimental.pallas import tpu as pltpu
from jax.experimental.pallas import tpu_sc as plsc

N = 10000
E = 320000
G = 64
D = 32

NC = 2
NS = 16
NW = NC * NS
CH = 128
PERW = 80
NCHT = NW * PERW
EPAD = NCHT * CH - E
RPT = 632
KB = 16
NP = RPT * NS


def _segsum_body(h_hbm, src_hbm, dst_hbm, zero_hbm, out_hbm,
                 acc_sh, src_v, dst_v, rows_a, gsem, sem):
    c = lax.axis_index("c")
    s = lax.axis_index("s")
    wid = c * NS + s
    cbase = wid * PERW

    pltpu.sync_copy(zero_hbm, acc_sh.at[pl.ds(s * RPT, RPT)])

    pltpu.sync_copy(src_hbm.at[pl.ds(cbase, PERW)], src_v)
    pltpu.sync_copy(dst_hbm.at[pl.ds(cbase, PERW)], dst_v)

    plsc.subcore_barrier()

    def group(j, carry):
        base = j * KB
        gds = [
            pltpu.async_copy(h_hbm.at[src_v.at[base + b]], rows_a.at[b],
                             gsem.at[b])
            for b in range(KB)
        ]
        sds = []
        for b in range(KB):
            gds[b].wait()
            sds.append(
                pltpu.async_copy(rows_a.at[b], acc_sh.at[dst_v.at[base + b]],
                                 sem, add=True))
        for sd in sds:
            sd.wait()
        return carry

    lax.fori_loop(0, PERW // KB, group, 0, unroll=False)

    plsc.subcore_barrier()

    pltpu.sync_copy(acc_sh.at[pl.ds(s * RPT, RPT)],
                    out_hbm.at[c, pl.ds(s * RPT, RPT)])


@functools.partial(jax.jit, static_argnums=())
def _segsum(h, src, dst, zero_blk):
    mesh = plsc.VectorSubcoreMesh(
        core_axis_name="c", subcore_axis_name="s",
        num_cores=NC, num_subcores=NS)
    fn = pl.kernel(
        _segsum_body,
        out_type=jax.ShapeDtypeStruct((NC, NP, D), jnp.float32),
        mesh=mesh,
        scratch_types=[
            pltpu.VMEM_SHARED((NP, D), jnp.float32),
            pltpu.VMEM((PERW, CH), jnp.int32),
            pltpu.VMEM((PERW, CH), jnp.int32),
            pltpu.VMEM((KB, CH, D), jnp.float32),
            pltpu.SemaphoreType.DMA((KB,)),
            pltpu.SemaphoreType.DMA,
        ],
        compiler_params=pltpu.CompilerParams(use_tc_tiling_on_sc=False),
    )
    return fn(h, src, dst, zero_blk)


def _proj_body(x_ref, w_ref, o_ref):
    o_ref[...] = jnp.dot(x_ref[...], w_ref[...],
                         preferred_element_type=jnp.float32)


def _proj(x, w, dout):
    return pl.pallas_call(
        _proj_body,
        out_shape=jax.ShapeDtypeStruct((x.shape[0], dout), jnp.float32),
    )(x, w)


def _norm_relu(u, g_ref, be_ref):
    mu = jnp.mean(u, axis=0, keepdims=True)
    var = jnp.mean((u - mu) ** 2, axis=0, keepdims=True)
    un = (u - mu) / jnp.sqrt(var + 1e-5) * g_ref[...] + be_ref[...]
    return jnp.maximum(un, 0.0)


def _mlp1_body(y_ref, agg_ref, eps_ref, ba_ref, g_ref, be_ref, wb_ref,
               bb_ref, o_ref):
    y = y_ref[...]
    u = (1.0 + eps_ref[0, 0]) * y + agg_ref[0, :N] + agg_ref[1, :N] + ba_ref[...]
    h = _norm_relu(u, g_ref, be_ref)
    o_ref[...] = jnp.dot(h, wb_ref[...],
                         preferred_element_type=jnp.float32) + bb_ref[...]


def _mlp1(y, agg, eps, ba, g, be, Wb, bb):
    return pl.pallas_call(
        _mlp1_body,
        out_shape=jax.ShapeDtypeStruct((N, Wb.shape[1]), jnp.float32),
    )(y, agg, eps.reshape(1, 1), ba.reshape(1, -1), g.reshape(1, -1),
      be.reshape(1, -1), Wb, bb.reshape(1, -1))


def _mlp_body(h_ref, agg_ref, eps_ref, wa_ref, ba_ref, g_ref, be_ref,
              wb_ref, bb_ref, o_ref):
    t = (1.0 + eps_ref[0, 0]) * h_ref[...] + agg_ref[0, :N] + agg_ref[1, :N]
    y = jnp.dot(t, wa_ref[...],
                preferred_element_type=jnp.float32) + ba_ref[...]
    h = _norm_relu(y, g_ref, be_ref)
    o_ref[...] = jnp.dot(h, wb_ref[...],
                         preferred_element_type=jnp.float32) + bb_ref[...]


def _mlp(h, agg, eps, Wa, ba, g, be, Wb, bb):
    return pl.pallas_call(
        _mlp_body,
        out_shape=jax.ShapeDtypeStruct((N, Wb.shape[1]), jnp.float32),
    )(h, agg, eps.reshape(1, 1), Wa, ba.reshape(1, -1), g.reshape(1, -1),
      be.reshape(1, -1), Wb, bb.reshape(1, -1))


def _final_body(h_ref, agg_ref, eps_ref, wa_ref, ba_ref, g_ref, be_ref,
                wb_ref, bb_ref, batch_ref, wl_ref, bl_ref, o_ref):
    t = (1.0 + eps_ref[0, 0]) * h_ref[...] + agg_ref[0, :N] + agg_ref[1, :N]
    y = jnp.dot(t, wa_ref[...],
                preferred_element_type=jnp.float32) + ba_ref[...]
    h = _norm_relu(y, g_ref, be_ref)
    h4 = jnp.dot(h, wb_ref[...],
                 preferred_element_type=jnp.float32) + bb_ref[...]
    gids = lax.broadcasted_iota(jnp.int32, (N, G), 1)
    onehot = (batch_ref[...] == gids).astype(jnp.float32)
    sums = lax.dot_general(onehot, h4, (((0,), (0,)), ((), ())),
                           preferred_element_type=jnp.float32)
    counts = jnp.sum(onehot, axis=0)[:, None]
    pooled = sums / jnp.maximum(counts, 1.0)
    logit = jnp.dot(pooled, wl_ref[...],
                    preferred_element_type=jnp.float32) + bl_ref[...]
    o_ref[...] = jax.nn.sigmoid(logit)


def _final(h, agg, eps, Wa, ba, g, be, Wb, bb, batch, Wl, bl):
    return pl.pallas_call(
        _final_body,
        out_shape=jax.ShapeDtypeStruct((G, 1), jnp.float32),
    )(h, agg, eps.reshape(1, 1), Wa, ba.reshape(1, -1), g.reshape(1, -1),
      be.reshape(1, -1), Wb, bb.reshape(1, -1), batch.reshape(N, 1), Wl,
      bl.reshape(1, -1))


def kernel(x, edge_index, batch, eps1, Wa1, ba1, g1, be1, Wb1, bb1,
           eps2, Wa2, ba2, g2, be2, Wb2, bb2,
           eps3, Wa3, ba3, g3, be3, Wb3, bb3,
           eps4, Wa4, ba4, g4, be4, Wb4, bb4, Wl, bl):
    pad_src = jnp.zeros((EPAD,), jnp.int32)
    pad_dst = jnp.full((EPAD,), N, jnp.int32)
    src = jnp.concatenate([edge_index[0], pad_src]).reshape(NCHT, CH)
    dst = jnp.concatenate([edge_index[1], pad_dst]).reshape(NCHT, CH)
    zero_blk = jnp.zeros((RPT, D), jnp.float32)

    y1 = _proj(x, Wa1, D)
    a1 = _segsum(y1, src, dst, zero_blk)
    h1 = _mlp1(y1, a1, eps1, ba1, g1, be1, Wb1, bb1)

    a2 = _segsum(h1, src, dst, zero_blk)
    h2 = _mlp(h1, a2, eps2, Wa2, ba2, g2, be2, Wb2, bb2)

    a3 = _segsum(h2, src, dst, zero_blk)
    h3 = _mlp(h2, a3, eps3, Wa3, ba3, g3, be3, Wb3, bb3)

    a4 = _segsum(h3, src, dst, zero_blk)
    return _final(h3, a4, eps4, Wa4, ba4, g4, be4, Wb4, bb4, batch, Wl, bl)


# speedup vs baseline: 16.1920x; 1.8256x over previous
import functools

import jax
import jax.numpy as jnp
from jax import lax
from jax.experimental import pallas as pl
from jax.experimental.pallas import tpu as pltpu
from jax.experimental.pallas import tpu_sc as plsc

N = 10000
E = 320000
G = 64
D = 32

NC = 2
NS = 16
NW = NC * NS
CH = 128
PERW = 80
NCHT = NW * PERW
EPAD = NCHT * CH - E
RPT = 632
KB = 16
NP = RPT * NS


def _segsum_body(h_hbm, src_hbm, dst_hbm, zero_hbm, out_hbm,
                 acc_sh, h_sh, src_v, dst_v, rows_a, gsem, sem):
    c = lax.axis_index("c")
    s = lax.axis_index("s")
    wid = c * NS + s
    cbase = wid * PERW

    pltpu.sync_copy(zero_hbm, acc_sh.at[pl.ds(s * RPT, RPT)])
    pltpu.sync_copy(h_hbm.at[pl.ds(s * RPT, RPT)],
                    h_sh.at[pl.ds(s * RPT, RPT)])

    pltpu.sync_copy(src_hbm.at[pl.ds(cbase, PERW)], src_v)
    pltpu.sync_copy(dst_hbm.at[pl.ds(cbase, PERW)], dst_v)

    plsc.subcore_barrier()

    def group(j, carry):
        base = j * KB
        gds = [
            pltpu.async_copy(h_sh.at[src_v.at[base + b]], rows_a.at[b],
                             gsem.at[b])
            for b in range(KB)
        ]
        sds = []
        for b in range(KB):
            gds[b].wait()
            sds.append(
                pltpu.async_copy(rows_a.at[b], acc_sh.at[dst_v.at[base + b]],
                                 sem, add=True))
        for sd in sds:
            sd.wait()
        return carry

    lax.fori_loop(0, PERW // KB, group, 0, unroll=False)

    plsc.subcore_barrier()

    pltpu.sync_copy(acc_sh.at[pl.ds(s * RPT, RPT)],
                    out_hbm.at[c, pl.ds(s * RPT, RPT)])


@functools.partial(jax.jit, static_argnums=())
def _segsum(h, src, dst, zero_blk):
    mesh = plsc.VectorSubcoreMesh(
        core_axis_name="c", subcore_axis_name="s",
        num_cores=NC, num_subcores=NS)
    fn = pl.kernel(
        _segsum_body,
        out_type=jax.ShapeDtypeStruct((NC, NP, D), jnp.float32),
        mesh=mesh,
        scratch_types=[
            pltpu.VMEM_SHARED((NP, D), jnp.float32),
            pltpu.VMEM_SHARED((NP, D), jnp.float32),
            pltpu.VMEM((PERW, CH), jnp.int32),
            pltpu.VMEM((PERW, CH), jnp.int32),
            pltpu.VMEM((KB, CH, D), jnp.float32),
            pltpu.SemaphoreType.DMA((KB,)),
            pltpu.SemaphoreType.DMA,
        ],
        compiler_params=pltpu.CompilerParams(use_tc_tiling_on_sc=False),
    )
    return fn(h, src, dst, zero_blk)


def _proj_body(x_ref, w_ref, o_ref):
    o_ref[:N] = jnp.dot(x_ref[...], w_ref[...],
                        preferred_element_type=jnp.float32)
    o_ref[N:] = jnp.zeros((NP - N, o_ref.shape[1]), jnp.float32)


def _proj(x, w, dout):
    return pl.pallas_call(
        _proj_body,
        out_shape=jax.ShapeDtypeStruct((NP, dout), jnp.float32),
    )(x, w)


def _norm_relu(u, g_ref, be_ref):
    mu = jnp.mean(u, axis=0, keepdims=True)
    var = jnp.mean((u - mu) ** 2, axis=0, keepdims=True)
    un = (u - mu) / jnp.sqrt(var + 1e-5) * g_ref[...] + be_ref[...]
    return jnp.maximum(un, 0.0)


def _mlp1_body(y_ref, agg_ref, eps_ref, ba_ref, g_ref, be_ref, wb_ref,
               bb_ref, o_ref):
    y = y_ref[:N]
    u = (1.0 + eps_ref[0, 0]) * y + agg_ref[0, :N] + agg_ref[1, :N] + ba_ref[...]
    h = _norm_relu(u, g_ref, be_ref)
    o_ref[:N] = jnp.dot(h, wb_ref[...],
                        preferred_element_type=jnp.float32) + bb_ref[...]
    o_ref[N:] = jnp.zeros((NP - N, o_ref.shape[1]), jnp.float32)


def _mlp1(y, agg, eps, ba, g, be, Wb, bb):
    return pl.pallas_call(
        _mlp1_body,
        out_shape=jax.ShapeDtypeStruct((NP, Wb.shape[1]), jnp.float32),
    )(y, agg, eps.reshape(1, 1), ba.reshape(1, -1), g.reshape(1, -1),
      be.reshape(1, -1), Wb, bb.reshape(1, -1))


def _mlp_body(h_ref, agg_ref, eps_ref, wa_ref, ba_ref, g_ref, be_ref,
              wb_ref, bb_ref, o_ref):
    t = (1.0 + eps_ref[0, 0]) * h_ref[:N] + agg_ref[0, :N] + agg_ref[1, :N]
    y = jnp.dot(t, wa_ref[...],
                preferred_element_type=jnp.float32) + ba_ref[...]
    h = _norm_relu(y, g_ref, be_ref)
    o_ref[:N] = jnp.dot(h, wb_ref[...],
                        preferred_element_type=jnp.float32) + bb_ref[...]
    o_ref[N:] = jnp.zeros((NP - N, o_ref.shape[1]), jnp.float32)


def _mlp(h, agg, eps, Wa, ba, g, be, Wb, bb):
    return pl.pallas_call(
        _mlp_body,
        out_shape=jax.ShapeDtypeStruct((NP, Wb.shape[1]), jnp.float32),
    )(h, agg, eps.reshape(1, 1), Wa, ba.reshape(1, -1), g.reshape(1, -1),
      be.reshape(1, -1), Wb, bb.reshape(1, -1))


def _final_body(h_ref, agg_ref, eps_ref, wa_ref, ba_ref, g_ref, be_ref,
                wb_ref, bb_ref, batch_ref, wl_ref, bl_ref, o_ref):
    t = (1.0 + eps_ref[0, 0]) * h_ref[:N] + agg_ref[0, :N] + agg_ref[1, :N]
    y = jnp.dot(t, wa_ref[...],
                preferred_element_type=jnp.float32) + ba_ref[...]
    h = _norm_relu(y, g_ref, be_ref)
    h4 = jnp.dot(h, wb_ref[...],
                 preferred_element_type=jnp.float32) + bb_ref[...]
    gids = lax.broadcasted_iota(jnp.int32, (N, G), 1)
    onehot = (batch_ref[...] == gids).astype(jnp.float32)
    sums = lax.dot_general(onehot, h4, (((0,), (0,)), ((), ())),
                           preferred_element_type=jnp.float32)
    counts = jnp.sum(onehot, axis=0)[:, None]
    pooled = sums / jnp.maximum(counts, 1.0)
    logit = jnp.dot(pooled, wl_ref[...],
                    preferred_element_type=jnp.float32) + bl_ref[...]
    o_ref[...] = jax.nn.sigmoid(logit)


def _final(h, agg, eps, Wa, ba, g, be, Wb, bb, batch, Wl, bl):
    return pl.pallas_call(
        _final_body,
        out_shape=jax.ShapeDtypeStruct((G, 1), jnp.float32),
    )(h, agg, eps.reshape(1, 1), Wa, ba.reshape(1, -1), g.reshape(1, -1),
      be.reshape(1, -1), Wb, bb.reshape(1, -1), batch.reshape(N, 1), Wl,
      bl.reshape(1, -1))


def kernel(x, edge_index, batch, eps1, Wa1, ba1, g1, be1, Wb1, bb1,
           eps2, Wa2, ba2, g2, be2, Wb2, bb2,
           eps3, Wa3, ba3, g3, be3, Wb3, bb3,
           eps4, Wa4, ba4, g4, be4, Wb4, bb4, Wl, bl):
    pad_src = jnp.zeros((EPAD,), jnp.int32)
    pad_dst = jnp.full((EPAD,), N, jnp.int32)
    src = jnp.concatenate([edge_index[0], pad_src]).reshape(NCHT, CH)
    dst = jnp.concatenate([edge_index[1], pad_dst]).reshape(NCHT, CH)
    zero_blk = jnp.zeros((RPT, D), jnp.float32)

    y1 = _proj(x, Wa1, D)
    a1 = _segsum(y1, src, dst, zero_blk)
    h1 = _mlp1(y1, a1, eps1, ba1, g1, be1, Wb1, bb1)

    a2 = _segsum(h1, src, dst, zero_blk)
    h2 = _mlp(h1, a2, eps2, Wa2, ba2, g2, be2, Wb2, bb2)

    a3 = _segsum(h2, src, dst, zero_blk)
    h3 = _mlp(h2, a3, eps3, Wa3, ba3, g3, be3, Wb3, bb3)

    a4 = _segsum(h3, src, dst, zero_blk)
    return _final(h3, a4, eps4, Wa4, ba4, g4, be4, Wb4, bb4, batch, Wl, bl)


# speedup vs baseline: 16.4642x; 1.0168x over previous
import functools

import jax
import jax.numpy as jnp
from jax import lax
from jax.experimental import pallas as pl
from jax.experimental.pallas import tpu as pltpu
from jax.experimental.pallas import tpu_sc as plsc

N = 10000
E = 320000
G = 64
D = 32

NC = 2
NS = 16
NW = NC * NS
CH = 128
PERW = 80
NCHT = NW * PERW
EPAD = NCHT * CH - E
RPT = 632
KB = 16
NP = RPT * NS


def _segsum_body(h_hbm, src_hbm, dst_hbm, out_hbm,
                 acc_sh, h_sh, src_v, dst_v, rows_a, zblk, gsem, sem):
    c = lax.axis_index("c")
    s = lax.axis_index("s")
    wid = c * NS + s
    cbase = wid * PERW

    pltpu.sync_copy(h_hbm.at[pl.ds(s * RPT, RPT)],
                    h_sh.at[pl.ds(s * RPT, RPT)])

    def zrow(i, carry):
        zblk[i, pl.ds(0, 16)] = jnp.zeros((16,), jnp.float32)
        zblk[i, pl.ds(16, 16)] = jnp.zeros((16,), jnp.float32)
        return carry

    lax.fori_loop(0, CH, zrow, 0, unroll=False)
    for r in range(4):
        pltpu.sync_copy(zblk, acc_sh.at[pl.ds(s * RPT + r * CH, CH)])
    pltpu.sync_copy(zblk.at[pl.ds(0, RPT - 4 * CH)],
                    acc_sh.at[pl.ds(s * RPT + 4 * CH, RPT - 4 * CH)])

    pltpu.sync_copy(src_hbm.at[pl.ds(cbase, PERW)], src_v)
    pltpu.sync_copy(dst_hbm.at[pl.ds(cbase, PERW)], dst_v)

    plsc.subcore_barrier()

    def group(j, carry):
        base = j * KB
        gds = [
            pltpu.async_copy(h_sh.at[src_v.at[base + b]], rows_a.at[b],
                             gsem.at[b])
            for b in range(KB)
        ]
        sds = []
        for b in range(KB):
            gds[b].wait()
            sds.append(
                pltpu.async_copy(rows_a.at[b], acc_sh.at[dst_v.at[base + b]],
                                 sem, add=True))
        for sd in sds:
            sd.wait()
        return carry

    lax.fori_loop(0, PERW // KB, group, 0, unroll=False)

    plsc.subcore_barrier()

    pltpu.sync_copy(acc_sh.at[pl.ds(s * RPT, RPT)],
                    out_hbm.at[c, pl.ds(s * RPT, RPT)])


@functools.partial(jax.jit, static_argnums=())
def _segsum(h, src, dst):
    mesh = plsc.VectorSubcoreMesh(
        core_axis_name="c", subcore_axis_name="s",
        num_cores=NC, num_subcores=NS)
    fn = pl.kernel(
        _segsum_body,
        out_type=jax.ShapeDtypeStruct((NC, NP, D), jnp.float32),
        mesh=mesh,
        scratch_types=[
            pltpu.VMEM_SHARED((NP, D), jnp.float32),
            pltpu.VMEM_SHARED((NP, D), jnp.float32),
            pltpu.VMEM((PERW, CH), jnp.int32),
            pltpu.VMEM((PERW, CH), jnp.int32),
            pltpu.VMEM((KB, CH, D), jnp.float32),
            pltpu.VMEM((CH, D), jnp.float32),
            pltpu.SemaphoreType.DMA((KB,)),
            pltpu.SemaphoreType.DMA,
        ],
        compiler_params=pltpu.CompilerParams(use_tc_tiling_on_sc=False),
    )
    return fn(h, src, dst)


def _proj_body(x_ref, w_ref, o_ref):
    o_ref[:N] = jnp.dot(x_ref[...], w_ref[...],
                        preferred_element_type=jnp.float32)
    o_ref[N:] = jnp.zeros((NP - N, o_ref.shape[1]), jnp.float32)


def _proj(x, w, dout):
    return pl.pallas_call(
        _proj_body,
        out_shape=jax.ShapeDtypeStruct((NP, dout), jnp.float32),
    )(x, w)


def _norm_relu(u, g_ref, be_ref):
    mu = jnp.mean(u, axis=0, keepdims=True)
    var = jnp.mean((u - mu) ** 2, axis=0, keepdims=True)
    un = (u - mu) / jnp.sqrt(var + 1e-5) * g_ref[...] + be_ref[...]
    return jnp.maximum(un, 0.0)


def _mlp1_body(y_ref, agg_ref, eps_ref, ba_ref, g_ref, be_ref, wb_ref,
               bb_ref, o_ref):
    y = y_ref[:N]
    u = (1.0 + eps_ref[0, 0]) * y + agg_ref[0, :N] + agg_ref[1, :N] + ba_ref[...]
    h = _norm_relu(u, g_ref, be_ref)
    o_ref[:N] = jnp.dot(h, wb_ref[...],
                        preferred_element_type=jnp.float32) + bb_ref[...]
    o_ref[N:] = jnp.zeros((NP - N, o_ref.shape[1]), jnp.float32)


def _mlp1(y, agg, eps, ba, g, be, Wb, bb):
    return pl.pallas_call(
        _mlp1_body,
        out_shape=jax.ShapeDtypeStruct((NP, Wb.shape[1]), jnp.float32),
    )(y, agg, eps.reshape(1, 1), ba.reshape(1, -1), g.reshape(1, -1),
      be.reshape(1, -1), Wb, bb.reshape(1, -1))


def _mlp_body(h_ref, agg_ref, eps_ref, wa_ref, ba_ref, g_ref, be_ref,
              wb_ref, bb_ref, o_ref):
    t = (1.0 + eps_ref[0, 0]) * h_ref[:N] + agg_ref[0, :N] + agg_ref[1, :N]
    y = jnp.dot(t, wa_ref[...],
                preferred_element_type=jnp.float32) + ba_ref[...]
    h = _norm_relu(y, g_ref, be_ref)
    o_ref[:N] = jnp.dot(h, wb_ref[...],
                        preferred_element_type=jnp.float32) + bb_ref[...]
    o_ref[N:] = jnp.zeros((NP - N, o_ref.shape[1]), jnp.float32)


def _mlp(h, agg, eps, Wa, ba, g, be, Wb, bb):
    return pl.pallas_call(
        _mlp_body,
        out_shape=jax.ShapeDtypeStruct((NP, Wb.shape[1]), jnp.float32),
    )(h, agg, eps.reshape(1, 1), Wa, ba.reshape(1, -1), g.reshape(1, -1),
      be.reshape(1, -1), Wb, bb.reshape(1, -1))


def _final_body(h_ref, agg_ref, eps_ref, wa_ref, ba_ref, g_ref, be_ref,
                wb_ref, bb_ref, batch_ref, wl_ref, bl_ref, o_ref):
    t = (1.0 + eps_ref[0, 0]) * h_ref[:N] + agg_ref[0, :N] + agg_ref[1, :N]
    y = jnp.dot(t, wa_ref[...],
                preferred_element_type=jnp.float32) + ba_ref[...]
    h = _norm_relu(y, g_ref, be_ref)
    h4 = jnp.dot(h, wb_ref[...],
                 preferred_element_type=jnp.float32) + bb_ref[...]
    gids = lax.broadcasted_iota(jnp.int32, (N, G), 1)
    onehot = (batch_ref[...] == gids).astype(jnp.float32)
    sums = lax.dot_general(onehot, h4, (((0,), (0,)), ((), ())),
                           preferred_element_type=jnp.float32)
    counts = jnp.sum(onehot, axis=0)[:, None]
    pooled = sums / jnp.maximum(counts, 1.0)
    logit = jnp.dot(pooled, wl_ref[...],
                    preferred_element_type=jnp.float32) + bl_ref[...]
    o_ref[...] = jax.nn.sigmoid(logit)


def _final(h, agg, eps, Wa, ba, g, be, Wb, bb, batch, Wl, bl):
    return pl.pallas_call(
        _final_body,
        out_shape=jax.ShapeDtypeStruct((G, 1), jnp.float32),
    )(h, agg, eps.reshape(1, 1), Wa, ba.reshape(1, -1), g.reshape(1, -1),
      be.reshape(1, -1), Wb, bb.reshape(1, -1), batch.reshape(N, 1), Wl,
      bl.reshape(1, -1))


def kernel(x, edge_index, batch, eps1, Wa1, ba1, g1, be1, Wb1, bb1,
           eps2, Wa2, ba2, g2, be2, Wb2, bb2,
           eps3, Wa3, ba3, g3, be3, Wb3, bb3,
           eps4, Wa4, ba4, g4, be4, Wb4, bb4, Wl, bl):
    pad_src = jnp.zeros((EPAD,), jnp.int32)
    pad_dst = jnp.full((EPAD,), N, jnp.int32)
    src = jnp.concatenate([edge_index[0], pad_src]).reshape(NCHT, CH)
    dst = jnp.concatenate([edge_index[1], pad_dst]).reshape(NCHT, CH)

    y1 = _proj(x, Wa1, D)
    a1 = _segsum(y1, src, dst)
    h1 = _mlp1(y1, a1, eps1, ba1, g1, be1, Wb1, bb1)

    a2 = _segsum(h1, src, dst)
    h2 = _mlp(h1, a2, eps2, Wa2, ba2, g2, be2, Wb2, bb2)

    a3 = _segsum(h2, src, dst)
    h3 = _mlp(h2, a3, eps3, Wa3, ba3, g3, be3, Wb3, bb3)

    a4 = _segsum(h3, src, dst)
    return _final(h3, a4, eps4, Wa4, ba4, g4, be4, Wb4, bb4, batch, Wl, bl)


# speedup vs baseline: 20.7929x; 1.2629x over previous
import functools

import jax
import jax.numpy as jnp
from jax import lax
from jax.experimental import pallas as pl
from jax.experimental.pallas import tpu as pltpu
from jax.experimental.pallas import tpu_sc as plsc

N = 10000
E = 320000
G = 64
D = 32

NC = 2
NS = 16
NW = NC * NS
CH = 128
PERW = 80
NCHT = NW * PERW
EPAD = NCHT * CH - E
RPT = 632
KB = 16
NP = RPT * NS


def _segsum_body(h_hbm, src_hbm, dst_hbm, out_hbm,
                 acc_sh, h_sh, src_v, dst_v, rows_a, zblk, gsem, sem):
    c = lax.axis_index("c")
    s = lax.axis_index("s")
    wid = c * NS + s
    cbase = wid * PERW

    pltpu.sync_copy(h_hbm.at[pl.ds(s * RPT, RPT)],
                    h_sh.at[pl.ds(s * RPT, RPT)])

    def zrow(i, carry):
        zblk[i, pl.ds(0, 16)] = jnp.zeros((16,), jnp.float32)
        zblk[i, pl.ds(16, 16)] = jnp.zeros((16,), jnp.float32)
        return carry

    lax.fori_loop(0, CH, zrow, 0, unroll=False)
    for r in range(4):
        pltpu.sync_copy(zblk, acc_sh.at[pl.ds(s * RPT + r * CH, CH)])
    pltpu.sync_copy(zblk.at[pl.ds(0, RPT - 4 * CH)],
                    acc_sh.at[pl.ds(s * RPT + 4 * CH, RPT - 4 * CH)])

    pltpu.sync_copy(src_hbm.at[pl.ds(cbase, PERW)], src_v)
    pltpu.sync_copy(dst_hbm.at[pl.ds(cbase, PERW)], dst_v)

    plsc.subcore_barrier()

    def group(j, carry):
        base = j * KB
        gds = [
            pltpu.async_copy(h_sh.at[src_v.at[base + b]], rows_a.at[b],
                             gsem.at[b])
            for b in range(KB)
        ]
        sds = []
        for b in range(KB):
            gds[b].wait()
            sds.append(
                pltpu.async_copy(rows_a.at[b], acc_sh.at[dst_v.at[base + b]],
                                 sem, add=True))
        for sd in sds:
            sd.wait()
        return carry

    lax.fori_loop(0, PERW // KB, group, 0, unroll=False)

    plsc.subcore_barrier()

    pltpu.sync_copy(acc_sh.at[pl.ds(s * RPT, RPT)],
                    out_hbm.at[c, pl.ds(s * RPT, RPT)])


@functools.partial(jax.jit, static_argnums=())
def _segsum(h, src, dst):
    mesh = plsc.VectorSubcoreMesh(
        core_axis_name="c", subcore_axis_name="s",
        num_cores=NC, num_subcores=NS)
    fn = pl.kernel(
        _segsum_body,
        out_type=jax.ShapeDtypeStruct((NC, NP, D), jnp.float32),
        mesh=mesh,
        scratch_types=[
            pltpu.VMEM_SHARED((NP, D), jnp.float32),
            pltpu.VMEM_SHARED((NP, D), jnp.float32),
            pltpu.VMEM((PERW, CH), jnp.int32),
            pltpu.VMEM((PERW, CH), jnp.int32),
            pltpu.VMEM((KB, CH, D), jnp.float32),
            pltpu.VMEM((CH, D), jnp.float32),
            pltpu.SemaphoreType.DMA((KB,)),
            pltpu.SemaphoreType.DMA,
        ],
        compiler_params=pltpu.CompilerParams(use_tc_tiling_on_sc=False),
    )
    return fn(h, src, dst)


PK = 4
PR = N // PK
PRP = NP // PK


def _fold_norm_relu(u, dh, g_t, be_t):
    L = u.shape[1]
    us = u[:PR]
    csum = jnp.sum(us, axis=0, keepdims=True)
    ii = lax.broadcasted_iota(jnp.int32, (L, L), 0) % dh
    jj = lax.broadcasted_iota(jnp.int32, (L, L), 1) % dh
    fold = (ii == jj).astype(jnp.float32)
    mu = jnp.dot(csum, fold, preferred_element_type=jnp.float32) / N
    d = u - mu
    ds = d[:PR]
    c2 = jnp.sum(ds * ds, axis=0, keepdims=True)
    var = jnp.dot(c2, fold, preferred_element_type=jnp.float32) / N
    return jnp.maximum(d / jnp.sqrt(var + 1e-5) * g_t + be_t, 0.0)


def _proj_body(x_ref, w_ref, o_ref):
    o_ref[...] = jnp.dot(x_ref[...], w_ref[...],
                         preferred_element_type=jnp.float32)


def _proj(x_pad, w_bd):
    return pl.pallas_call(
        _proj_body,
        out_shape=jax.ShapeDtypeStruct((PRP, PK * D), jnp.float32),
    )(x_pad, w_bd)


def _mlp1_body(y_ref, agg_ref, eps_ref, ba_ref, g_ref, be_ref, wb_ref,
               bb_ref, o_ref):
    u = ((1.0 + eps_ref[0, 0]) * y_ref[...] + agg_ref[0] + agg_ref[1]
         + ba_ref[...])
    h = _fold_norm_relu(u, D, g_ref[...], be_ref[...])
    o_ref[...] = jnp.dot(h, wb_ref[...],
                         preferred_element_type=jnp.float32) + bb_ref[...]


def _mlp1(y, agg, eps, ba_t, g_t, be_t, wb_bd, bb_t):
    return pl.pallas_call(
        _mlp1_body,
        out_shape=jax.ShapeDtypeStruct((PRP, PK * D), jnp.float32),
    )(y, agg, eps.reshape(1, 1), ba_t, g_t, be_t, wb_bd, bb_t)


def _mlp_body(h_ref, agg_ref, eps_ref, wa_ref, ba_ref, g_ref, be_ref,
              wb_ref, bb_ref, o_ref):
    t = (1.0 + eps_ref[0, 0]) * h_ref[...] + agg_ref[0] + agg_ref[1]
    y = jnp.dot(t, wa_ref[...],
                preferred_element_type=jnp.float32) + ba_ref[...]
    h = _fold_norm_relu(y, 64, g_ref[...], be_ref[...])
    o_ref[...] = jnp.dot(h, wb_ref[...],
                         preferred_element_type=jnp.float32) + bb_ref[...]


def _mlp(h, agg, eps, wa_bd, ba_t, g_t, be_t, wb_bd, bb_t):
    return pl.pallas_call(
        _mlp_body,
        out_shape=jax.ShapeDtypeStruct((PRP, PK * D), jnp.float32),
    )(h, agg, eps.reshape(1, 1), wa_bd, ba_t, g_t, be_t, wb_bd, bb_t)


def _final_body(h_ref, agg_ref, eps_ref, wa_ref, ba_ref, g_ref, be_ref,
                wb_ref, bb_ref, batch_ref, wl_ref, bl_ref, o_ref):
    t = (1.0 + eps_ref[0, 0]) * h_ref[...] + agg_ref[0] + agg_ref[1]
    y = jnp.dot(t, wa_ref[...],
                preferred_element_type=jnp.float32) + ba_ref[...]
    h = _fold_norm_relu(y, 64, g_ref[...], be_ref[...])
    h4 = jnp.dot(h, wb_ref[...],
                 preferred_element_type=jnp.float32) + bb_ref[...]
    gids = lax.broadcasted_iota(jnp.int32, (PR, G), 1)
    sums = jnp.zeros((G, 16), jnp.float32)
    counts = jnp.zeros((G, 1), jnp.float32)
    for k in range(PK):
        oh = (batch_ref[:, k:k + 1] == gids).astype(jnp.float32)
        sums = sums + lax.dot_general(
            oh, h4[:PR, 16 * k:16 * k + 16], (((0,), (0,)), ((), ())),
            preferred_element_type=jnp.float32)
        counts = counts + jnp.sum(oh, axis=0)[:, None]
    pooled = sums / jnp.maximum(counts, 1.0)
    logit = jnp.dot(pooled, wl_ref[...],
                    preferred_element_type=jnp.float32) + bl_ref[...]
    o_ref[...] = jax.nn.sigmoid(logit)


def _final(h, agg, eps, wa_bd, ba_t, g_t, be_t, wb_bd, bb_t, batch_p,
           Wl, bl):
    return pl.pallas_call(
        _final_body,
        out_shape=jax.ShapeDtypeStruct((G, 1), jnp.float32),
    )(h, agg, eps.reshape(1, 1), wa_bd, ba_t, g_t, be_t, wb_bd, bb_t,
      batch_p, Wl, bl.reshape(1, 1))


def _bd(W):
    return jnp.kron(jnp.eye(PK, dtype=jnp.float32), W)


def _t4(v):
    return jnp.tile(v, PK)[None, :]


def kernel(x, edge_index, batch, eps1, Wa1, ba1, g1, be1, Wb1, bb1,
           eps2, Wa2, ba2, g2, be2, Wb2, bb2,
           eps3, Wa3, ba3, g3, be3, Wb3, bb3,
           eps4, Wa4, ba4, g4, be4, Wb4, bb4, Wl, bl):
    pad_src = jnp.zeros((EPAD,), jnp.int32)
    pad_dst = jnp.full((EPAD,), N, jnp.int32)
    src = jnp.concatenate([edge_index[0], pad_src]).reshape(NCHT, CH)
    dst = jnp.concatenate([edge_index[1], pad_dst]).reshape(NCHT, CH)
    x_pad = jnp.concatenate(
        [x, jnp.zeros((NP - N, x.shape[1]), jnp.float32)]).reshape(PRP, -1)
    batch_p = batch.reshape(PR, PK)

    y1 = _proj(x_pad, _bd(Wa1))
    a1 = _segsum(y1.reshape(NP, D), src, dst)
    h1 = _mlp1(y1, a1.reshape(NC, PRP, PK * D), eps1, _t4(ba1), _t4(g1),
               _t4(be1), _bd(Wb1), _t4(bb1))

    a2 = _segsum(h1.reshape(NP, D), src, dst)
    h2 = _mlp(h1, a2.reshape(NC, PRP, PK * D), eps2, _bd(Wa2), _t4(ba2),
              _t4(g2), _t4(be2), _bd(Wb2), _t4(bb2))

    a3 = _segsum(h2.reshape(NP, D), src, dst)
    h3 = _mlp(h2, a3.reshape(NC, PRP, PK * D), eps3, _bd(Wa3), _t4(ba3),
              _t4(g3), _t4(be3), _bd(Wb3), _t4(bb3))

    a4 = _segsum(h3.reshape(NP, D), src, dst)
    return _final(h3, a4.reshape(NC, PRP, PK * D), eps4, _bd(Wa4),
                  _t4(ba4), _t4(g4), _t4(be4), _bd(Wb4), _t4(bb4),
                  batch_p, Wl, bl)


# speedup vs baseline: 21.8605x; 1.0513x over previous
import functools

import jax
import jax.numpy as jnp
from jax import lax
from jax.experimental import pallas as pl
from jax.experimental.pallas import tpu as pltpu
from jax.experimental.pallas import tpu_sc as plsc

N = 10000
E = 320000
G = 64
D = 32

NC = 2
NS = 16
NW = NC * NS
CH = 128
PERW = 80
NCHT = NW * PERW
EPAD = NCHT * CH - E
RPT = 632
KB = 16
NP = RPT * NS


def _segsum_body(h_hbm, src_hbm, dst_hbm, out_hbm,
                 acc_sh, h_sh, src_v, dst_v, rows_a, zblk, gsem, psem, sem):
    c = lax.axis_index("c")
    s = lax.axis_index("s")
    wid = c * NS + s
    cbase = wid * PERW

    hd = pltpu.async_copy(h_hbm.at[pl.ds(s * RPT, RPT)],
                          h_sh.at[pl.ds(s * RPT, RPT)], psem.at[0])
    sd = pltpu.async_copy(src_hbm.at[pl.ds(cbase, PERW)], src_v, psem.at[1])
    dd = pltpu.async_copy(dst_hbm.at[pl.ds(cbase, PERW)], dst_v, psem.at[2])

    def zrow(i, carry):
        zblk[i, pl.ds(0, 16)] = jnp.zeros((16,), jnp.float32)
        zblk[i, pl.ds(16, 16)] = jnp.zeros((16,), jnp.float32)
        return carry

    lax.fori_loop(0, CH, zrow, 0, unroll=False)
    for r in range(4):
        pltpu.sync_copy(zblk, acc_sh.at[pl.ds(s * RPT + r * CH, CH)])
    pltpu.sync_copy(zblk.at[pl.ds(0, RPT - 4 * CH)],
                    acc_sh.at[pl.ds(s * RPT + 4 * CH, RPT - 4 * CH)])
    hd.wait()
    sd.wait()
    dd.wait()

    plsc.subcore_barrier()

    def group(j, carry):
        base = j * KB
        gds = [
            pltpu.async_copy(h_sh.at[src_v.at[base + b]], rows_a.at[b],
                             gsem.at[b])
            for b in range(KB)
        ]
        sds = []
        for b in range(KB):
            gds[b].wait()
            sds.append(
                pltpu.async_copy(rows_a.at[b], acc_sh.at[dst_v.at[base + b]],
                                 sem, add=True))
        for sd in sds:
            sd.wait()
        return carry

    lax.fori_loop(0, PERW // KB, group, 0, unroll=False)

    plsc.subcore_barrier()

    pltpu.sync_copy(acc_sh.at[pl.ds(s * RPT, RPT)],
                    out_hbm.at[c, pl.ds(s * RPT, RPT)])


@functools.partial(jax.jit, static_argnums=())
def _segsum(h, src, dst):
    mesh = plsc.VectorSubcoreMesh(
        core_axis_name="c", subcore_axis_name="s",
        num_cores=NC, num_subcores=NS)
    fn = pl.kernel(
        _segsum_body,
        out_type=jax.ShapeDtypeStruct((NC, NP, D), jnp.float32),
        mesh=mesh,
        scratch_types=[
            pltpu.VMEM_SHARED((NP, D), jnp.float32),
            pltpu.VMEM_SHARED((NP, D), jnp.float32),
            pltpu.VMEM((PERW, CH), jnp.int32),
            pltpu.VMEM((PERW, CH), jnp.int32),
            pltpu.VMEM((KB, CH, D), jnp.float32),
            pltpu.VMEM((CH, D), jnp.float32),
            pltpu.SemaphoreType.DMA((KB,)),
            pltpu.SemaphoreType.DMA((3,)),
            pltpu.SemaphoreType.DMA,
        ],
        compiler_params=pltpu.CompilerParams(use_tc_tiling_on_sc=False),
    )
    return fn(h, src, dst)


PK = 4
PR = N // PK
PRP = NP // PK


def _fold_norm_relu(u, dh, g_t, be_t):
    L = u.shape[1]
    us = u[:PR]
    csum = jnp.sum(us, axis=0, keepdims=True)
    ii = lax.broadcasted_iota(jnp.int32, (L, L), 0) % dh
    jj = lax.broadcasted_iota(jnp.int32, (L, L), 1) % dh
    fold = (ii == jj).astype(jnp.float32)
    mu = jnp.dot(csum, fold, preferred_element_type=jnp.float32) / N
    d = u - mu
    ds = d[:PR]
    c2 = jnp.sum(ds * ds, axis=0, keepdims=True)
    var = jnp.dot(c2, fold, preferred_element_type=jnp.float32) / N
    return jnp.maximum(d / jnp.sqrt(var + 1e-5) * g_t + be_t, 0.0)


def _proj_body(x_ref, w_ref, o_ref):
    o_ref[...] = jnp.dot(x_ref[...], w_ref[...],
                         preferred_element_type=jnp.float32)


def _proj(x_pad, w_bd):
    return pl.pallas_call(
        _proj_body,
        out_shape=jax.ShapeDtypeStruct((PRP, PK * D), jnp.float32),
    )(x_pad, w_bd)


def _mlp1_body(y_ref, agg_ref, eps_ref, ba_ref, g_ref, be_ref, wb_ref,
               bb_ref, o_ref):
    u = ((1.0 + eps_ref[0, 0]) * y_ref[...] + agg_ref[0] + agg_ref[1]
         + ba_ref[...])
    h = _fold_norm_relu(u, D, g_ref[...], be_ref[...])
    o_ref[...] = jnp.dot(h, wb_ref[...],
                         preferred_element_type=jnp.float32) + bb_ref[...]


def _mlp1(y, agg, eps, ba_t, g_t, be_t, wb_bd, bb_t):
    return pl.pallas_call(
        _mlp1_body,
        out_shape=jax.ShapeDtypeStruct((PRP, PK * D), jnp.float32),
    )(y, agg, eps.reshape(1, 1), ba_t, g_t, be_t, wb_bd, bb_t)


def _mlp_body(h_ref, agg_ref, eps_ref, wa_ref, ba_ref, g_ref, be_ref,
              wb_ref, bb_ref, o_ref):
    t = (1.0 + eps_ref[0, 0]) * h_ref[...] + agg_ref[0] + agg_ref[1]
    y = jnp.dot(t, wa_ref[...],
                preferred_element_type=jnp.float32) + ba_ref[...]
    h = _fold_norm_relu(y, 64, g_ref[...], be_ref[...])
    o_ref[...] = jnp.dot(h, wb_ref[...],
                         preferred_element_type=jnp.float32) + bb_ref[...]


def _mlp(h, agg, eps, wa_bd, ba_t, g_t, be_t, wb_bd, bb_t):
    return pl.pallas_call(
        _mlp_body,
        out_shape=jax.ShapeDtypeStruct((PRP, PK * D), jnp.float32),
    )(h, agg, eps.reshape(1, 1), wa_bd, ba_t, g_t, be_t, wb_bd, bb_t)


def _final_body(h_ref, agg_ref, eps_ref, wa_ref, ba_ref, g_ref, be_ref,
                wb_ref, bb_ref, batch_ref, wl_ref, bl_ref, o_ref):
    t = (1.0 + eps_ref[0, 0]) * h_ref[...] + agg_ref[0] + agg_ref[1]
    y = jnp.dot(t, wa_ref[...],
                preferred_element_type=jnp.float32) + ba_ref[...]
    h = _fold_norm_relu(y, 64, g_ref[...], be_ref[...])
    h4 = jnp.dot(h, wb_ref[...],
                 preferred_element_type=jnp.float32) + bb_ref[...]
    gids = lax.broadcasted_iota(jnp.int32, (PR, G), 1)
    sums = jnp.zeros((G, 16), jnp.float32)
    counts = jnp.zeros((G, 1), jnp.float32)
    for k in range(PK):
        oh = (batch_ref[:, k:k + 1] == gids).astype(jnp.float32)
        sums = sums + lax.dot_general(
            oh, h4[:PR, 16 * k:16 * k + 16], (((0,), (0,)), ((), ())),
            preferred_element_type=jnp.float32)
        counts = counts + jnp.sum(oh, axis=0)[:, None]
    pooled = sums / jnp.maximum(counts, 1.0)
    logit = jnp.dot(pooled, wl_ref[...],
                    preferred_element_type=jnp.float32) + bl_ref[...]
    o_ref[...] = jax.nn.sigmoid(logit)


def _final(h, agg, eps, wa_bd, ba_t, g_t, be_t, wb_bd, bb_t, batch_p,
           Wl, bl):
    return pl.pallas_call(
        _final_body,
        out_shape=jax.ShapeDtypeStruct((G, 1), jnp.float32),
    )(h, agg, eps.reshape(1, 1), wa_bd, ba_t, g_t, be_t, wb_bd, bb_t,
      batch_p, Wl, bl.reshape(1, 1))


def _bd(W):
    return jnp.kron(jnp.eye(PK, dtype=jnp.float32), W)


def _t4(v):
    return jnp.tile(v, PK)[None, :]


def kernel(x, edge_index, batch, eps1, Wa1, ba1, g1, be1, Wb1, bb1,
           eps2, Wa2, ba2, g2, be2, Wb2, bb2,
           eps3, Wa3, ba3, g3, be3, Wb3, bb3,
           eps4, Wa4, ba4, g4, be4, Wb4, bb4, Wl, bl):
    pad_src = jnp.zeros((EPAD,), jnp.int32)
    pad_dst = jnp.full((EPAD,), N, jnp.int32)
    src = jnp.concatenate([edge_index[0], pad_src]).reshape(NCHT, CH)
    dst = jnp.concatenate([edge_index[1], pad_dst]).reshape(NCHT, CH)
    x_pad = jnp.concatenate(
        [x, jnp.zeros((NP - N, x.shape[1]), jnp.float32)]).reshape(PRP, -1)
    batch_p = batch.reshape(PR, PK)

    y1 = _proj(x_pad, _bd(Wa1))
    a1 = _segsum(y1.reshape(NP, D), src, dst)
    h1 = _mlp1(y1, a1.reshape(NC, PRP, PK * D), eps1, _t4(ba1), _t4(g1),
               _t4(be1), _bd(Wb1), _t4(bb1))

    a2 = _segsum(h1.reshape(NP, D), src, dst)
    h2 = _mlp(h1, a2.reshape(NC, PRP, PK * D), eps2, _bd(Wa2), _t4(ba2),
              _t4(g2), _t4(be2), _bd(Wb2), _t4(bb2))

    a3 = _segsum(h2.reshape(NP, D), src, dst)
    h3 = _mlp(h2, a3.reshape(NC, PRP, PK * D), eps3, _bd(Wa3), _t4(ba3),
              _t4(g3), _t4(be3), _bd(Wb3), _t4(bb3))

    a4 = _segsum(h3.reshape(NP, D), src, dst)
    return _final(h3, a4.reshape(NC, PRP, PK * D), eps4, _bd(Wa4),
                  _t4(ba4), _t4(g4), _t4(be4), _bd(Wb4), _t4(bb4),
                  batch_p, Wl, bl)


# speedup vs baseline: 22.5613x; 1.0321x over previous
import functools

import jax
import jax.numpy as jnp
from jax import lax
from jax.experimental import pallas as pl
from jax.experimental.pallas import tpu as pltpu
from jax.experimental.pallas import tpu_sc as plsc

N = 10000
E = 320000
G = 64
D = 32

NC = 2
NS = 16
NW = NC * NS
CH = 128
PERW = 80
NCHT = NW * PERW
EPAD = NCHT * CH - E
RPT = 632
KB = 16
HB = KB // 2
NP = RPT * NS


def _segsum_body(h_hbm, src_hbm, dst_hbm, out_hbm,
                 acc_sh, h_sh, src_v, dst_v, rows_a, zblk, gsem, psem, sem):
    c = lax.axis_index("c")
    s = lax.axis_index("s")
    wid = c * NS + s
    cbase = wid * PERW

    hd = pltpu.async_copy(h_hbm.at[pl.ds(s * RPT, RPT)],
                          h_sh.at[pl.ds(s * RPT, RPT)], psem.at[0])
    sd = pltpu.async_copy(src_hbm.at[pl.ds(cbase, PERW)], src_v, psem.at[1])
    dd = pltpu.async_copy(dst_hbm.at[pl.ds(cbase, PERW)], dst_v, psem.at[2])

    def zrow(i, carry):
        zblk[i, pl.ds(0, 16)] = jnp.zeros((16,), jnp.float32)
        zblk[i, pl.ds(16, 16)] = jnp.zeros((16,), jnp.float32)
        return carry

    lax.fori_loop(0, CH, zrow, 0, unroll=False)
    for r in range(4):
        pltpu.sync_copy(zblk, acc_sh.at[pl.ds(s * RPT + r * CH, CH)])
    pltpu.sync_copy(zblk.at[pl.ds(0, RPT - 4 * CH)],
                    acc_sh.at[pl.ds(s * RPT + 4 * CH, RPT - 4 * CH)])
    hd.wait()
    sd.wait()
    dd.wait()

    plsc.subcore_barrier()

    def fire_half(g, off):
        for b in range(HB):
            pltpu.async_copy(h_sh.at[src_v.at[g * HB + b]],
                             rows_a.at[off + b], gsem.at[off + b])

    def drain_half(g, off):
        sds = []
        for b in range(HB):
            pltpu.make_async_copy(h_sh.at[src_v.at[g * HB + b]],
                                  rows_a.at[off + b],
                                  gsem.at[off + b]).wait()
            sds.append(
                pltpu.async_copy(rows_a.at[off + b],
                                 acc_sh.at[dst_v.at[g * HB + b]],
                                 sem, add=True))
        return sds

    NG2 = PERW // KB
    fire_half(0, 0)

    def ring(j, carry):
        fire_half(2 * j + 1, HB)
        for sd in drain_half(2 * j, 0):
            sd.wait()

        @pl.when(j < NG2 - 1)
        def _():
            fire_half(2 * j + 2, 0)

        for sd in drain_half(2 * j + 1, HB):
            sd.wait()
        return carry

    lax.fori_loop(0, NG2, ring, 0, unroll=False)

    plsc.subcore_barrier()

    pltpu.sync_copy(acc_sh.at[pl.ds(s * RPT, RPT)],
                    out_hbm.at[c, pl.ds(s * RPT, RPT)])


@functools.partial(jax.jit, static_argnums=())
def _segsum(h, src, dst):
    mesh = plsc.VectorSubcoreMesh(
        core_axis_name="c", subcore_axis_name="s",
        num_cores=NC, num_subcores=NS)
    fn = pl.kernel(
        _segsum_body,
        out_type=jax.ShapeDtypeStruct((NC, NP, D), jnp.float32),
        mesh=mesh,
        scratch_types=[
            pltpu.VMEM_SHARED((NP, D), jnp.float32),
            pltpu.VMEM_SHARED((NP, D), jnp.float32),
            pltpu.VMEM((PERW, CH), jnp.int32),
            pltpu.VMEM((PERW, CH), jnp.int32),
            pltpu.VMEM((KB, CH, D), jnp.float32),
            pltpu.VMEM((CH, D), jnp.float32),
            pltpu.SemaphoreType.DMA((KB,)),
            pltpu.SemaphoreType.DMA((3,)),
            pltpu.SemaphoreType.DMA,
        ],
        compiler_params=pltpu.CompilerParams(use_tc_tiling_on_sc=False),
    )
    return fn(h, src, dst)


PK = 4
PR = N // PK
PRP = NP // PK


def _fold_norm_relu(u, dh, g_t, be_t):
    L = u.shape[1]
    us = u[:PR]
    csum = jnp.sum(us, axis=0, keepdims=True)
    ii = lax.broadcasted_iota(jnp.int32, (L, L), 0) % dh
    jj = lax.broadcasted_iota(jnp.int32, (L, L), 1) % dh
    fold = (ii == jj).astype(jnp.float32)
    mu = jnp.dot(csum, fold, preferred_element_type=jnp.float32) / N
    d = u - mu
    ds = d[:PR]
    c2 = jnp.sum(ds * ds, axis=0, keepdims=True)
    var = jnp.dot(c2, fold, preferred_element_type=jnp.float32) / N
    return jnp.maximum(d / jnp.sqrt(var + 1e-5) * g_t + be_t, 0.0)


def _proj_body(x_ref, w_ref, o_ref):
    o_ref[...] = jnp.dot(x_ref[...], w_ref[...],
                         preferred_element_type=jnp.float32)


def _proj(x_pad, w_bd):
    return pl.pallas_call(
        _proj_body,
        out_shape=jax.ShapeDtypeStruct((PRP, PK * D), jnp.float32),
    )(x_pad, w_bd)


def _mlp1_body(y_ref, agg_ref, eps_ref, ba_ref, g_ref, be_ref, wb_ref,
               bb_ref, o_ref):
    u = ((1.0 + eps_ref[0, 0]) * y_ref[...] + agg_ref[0] + agg_ref[1]
         + ba_ref[...])
    h = _fold_norm_relu(u, D, g_ref[...], be_ref[...])
    o_ref[...] = jnp.dot(h, wb_ref[...],
                         preferred_element_type=jnp.float32) + bb_ref[...]


def _mlp1(y, agg, eps, ba_t, g_t, be_t, wb_bd, bb_t):
    return pl.pallas_call(
        _mlp1_body,
        out_shape=jax.ShapeDtypeStruct((PRP, PK * D), jnp.float32),
    )(y, agg, eps.reshape(1, 1), ba_t, g_t, be_t, wb_bd, bb_t)


def _mlp_body(h_ref, agg_ref, eps_ref, wa_ref, ba_ref, g_ref, be_ref,
              wb_ref, bb_ref, o_ref):
    t = (1.0 + eps_ref[0, 0]) * h_ref[...] + agg_ref[0] + agg_ref[1]
    y = jnp.dot(t, wa_ref[...],
                preferred_element_type=jnp.float32) + ba_ref[...]
    h = _fold_norm_relu(y, 64, g_ref[...], be_ref[...])
    o_ref[...] = jnp.dot(h, wb_ref[...],
                         preferred_element_type=jnp.float32) + bb_ref[...]


def _mlp(h, agg, eps, wa_bd, ba_t, g_t, be_t, wb_bd, bb_t):
    return pl.pallas_call(
        _mlp_body,
        out_shape=jax.ShapeDtypeStruct((PRP, PK * D), jnp.float32),
    )(h, agg, eps.reshape(1, 1), wa_bd, ba_t, g_t, be_t, wb_bd, bb_t)


def _final_body(h_ref, agg_ref, eps_ref, wa_ref, ba_ref, g_ref, be_ref,
                wb_ref, bb_ref, batch_ref, wl_ref, bl_ref, o_ref):
    t = (1.0 + eps_ref[0, 0]) * h_ref[...] + agg_ref[0] + agg_ref[1]
    y = jnp.dot(t, wa_ref[...],
                preferred_element_type=jnp.float32) + ba_ref[...]
    h = _fold_norm_relu(y, 64, g_ref[...], be_ref[...])
    h4 = jnp.dot(h, wb_ref[...],
                 preferred_element_type=jnp.float32) + bb_ref[...]
    gids = lax.broadcasted_iota(jnp.int32, (PR, G), 1)
    sums = jnp.zeros((G, 16), jnp.float32)
    counts = jnp.zeros((G, 1), jnp.float32)
    for k in range(PK):
        oh = (batch_ref[:, k:k + 1] == gids).astype(jnp.float32)
        sums = sums + lax.dot_general(
            oh, h4[:PR, 16 * k:16 * k + 16], (((0,), (0,)), ((), ())),
            preferred_element_type=jnp.float32)
        counts = counts + jnp.sum(oh, axis=0)[:, None]
    pooled = sums / jnp.maximum(counts, 1.0)
    logit = jnp.dot(pooled, wl_ref[...],
                    preferred_element_type=jnp.float32) + bl_ref[...]
    o_ref[...] = jax.nn.sigmoid(logit)


def _final(h, agg, eps, wa_bd, ba_t, g_t, be_t, wb_bd, bb_t, batch_p,
           Wl, bl):
    return pl.pallas_call(
        _final_body,
        out_shape=jax.ShapeDtypeStruct((G, 1), jnp.float32),
    )(h, agg, eps.reshape(1, 1), wa_bd, ba_t, g_t, be_t, wb_bd, bb_t,
      batch_p, Wl, bl.reshape(1, 1))


def _bd(W):
    return jnp.kron(jnp.eye(PK, dtype=jnp.float32), W)


def _t4(v):
    return jnp.tile(v, PK)[None, :]


def kernel(x, edge_index, batch, eps1, Wa1, ba1, g1, be1, Wb1, bb1,
           eps2, Wa2, ba2, g2, be2, Wb2, bb2,
           eps3, Wa3, ba3, g3, be3, Wb3, bb3,
           eps4, Wa4, ba4, g4, be4, Wb4, bb4, Wl, bl):
    pad_src = jnp.zeros((EPAD,), jnp.int32)
    pad_dst = jnp.full((EPAD,), N, jnp.int32)
    src = jnp.concatenate([edge_index[0], pad_src]).reshape(NCHT, CH)
    dst = jnp.concatenate([edge_index[1], pad_dst]).reshape(NCHT, CH)
    x_pad = jnp.concatenate(
        [x, jnp.zeros((NP - N, x.shape[1]), jnp.float32)]).reshape(PRP, -1)
    batch_p = batch.reshape(PR, PK)

    y1 = _proj(x_pad, _bd(Wa1))
    a1 = _segsum(y1.reshape(NP, D), src, dst)
    h1 = _mlp1(y1, a1.reshape(NC, PRP, PK * D), eps1, _t4(ba1), _t4(g1),
               _t4(be1), _bd(Wb1), _t4(bb1))

    a2 = _segsum(h1.reshape(NP, D), src, dst)
    h2 = _mlp(h1, a2.reshape(NC, PRP, PK * D), eps2, _bd(Wa2), _t4(ba2),
              _t4(g2), _t4(be2), _bd(Wb2), _t4(bb2))

    a3 = _segsum(h2.reshape(NP, D), src, dst)
    h3 = _mlp(h2, a3.reshape(NC, PRP, PK * D), eps3, _bd(Wa3), _t4(ba3),
              _t4(g3), _t4(be3), _bd(Wb3), _t4(bb3))

    a4 = _segsum(h3.reshape(NP, D), src, dst)
    return _final(h3, a4.reshape(NC, PRP, PK * D), eps4, _bd(Wa4),
                  _t4(ba4), _t4(g4), _t4(be4), _bd(Wb4), _t4(bb4),
                  batch_p, Wl, bl)
